# Initial kernel scaffold; baseline (speedup 1.0000x reference)
import jax
import jax.numpy as jnp
from jax.experimental import pallas as pl


def kernel(protein_x, protein_edge_index, ligand_x, ligand_edge_index, ligand_edge_attr, W_atom, b_atom, W_bond, b_bond, W_gat, a_src, a_dst, W_msg, b_msg, W_upd, b_upd, W_int, b_int, W_kd, b_kd):
    raise NotImplementedError("write your pallas kernel here")


# speedup vs baseline: 4.8125x; 4.8125x over previous
import functools

import jax
import jax.numpy as jnp
from jax import lax
from jax.experimental import pallas as pl
from jax.experimental.pallas import tpu as pltpu
from jax.experimental.pallas import tpu_sc as plsc

N = 10000
D = 128
D_EDGE = 16
NC, NS, L = 2, 16, 16
NW = NC * NS
CH = 128
NACC = 10016
DUMMY = N

E_P = 320000
E_P_PAD = 327680
EPT_P = E_P_PAD // NW
E_L = 160000
E_L_PAD = 163840
EPT_L = E_L_PAD // NW

_HI = lax.Precision.HIGHEST


def _dot(a, b):
    return lax.dot_general(a, b, (((1,), (0,)), ((), ())), precision=_HI)


_MESH = plsc.VectorSubcoreMesh(core_axis_name="c", subcore_axis_name="s")
_SC_PARAMS = pltpu.CompilerParams(needs_layout_passes=False,
                                  use_tc_tiling_on_sc=False)


def _gat_sc_body(src_hbm, dst_hbm, s_hbm, d_hbm, h_hbm, zeros_hbm,
                 w_out, denom_out,
                 s_vm, d_vm, denom_vm, sidx, didx, ex_vm, rows_vm,
                 acc_sh, sem):
    c = lax.axis_index("c")
    s = lax.axis_index("s")
    wid = c * NS + s

    @pl.when(s == 0)
    def _():
        pltpu.sync_copy(zeros_hbm, acc_sh.at[pl.ds(0, N)])

    pltpu.sync_copy(s_hbm, s_vm.at[pl.ds(0, N)])
    pltpu.sync_copy(d_hbm, d_vm.at[pl.ds(0, N)])
    s_vm[pl.ds(N, L)] = jnp.zeros((L,), jnp.float32)
    d_vm[pl.ds(N, L)] = jnp.zeros((L,), jnp.float32)

    def _zero(i, carry):
        denom_vm[pl.ds(i * L, L)] = jnp.zeros((L,), jnp.float32)
        return carry

    lax.fori_loop(0, NACC // L, _zero, 0)
    plsc.subcore_barrier()

    base_e = wid * EPT_P

    def _chunk(g, carry):
        eb = pl.multiple_of(base_e + g * CH, CH)
        pltpu.sync_copy(src_hbm.at[pl.ds(eb, CH)], sidx)
        pltpu.sync_copy(dst_hbm.at[pl.ds(eb, CH)], didx)
        gather = pltpu.async_copy(h_hbm.at[sidx], rows_vm, sem)

        def _v16(i, carry2):
            sl = pl.ds(i * L, L)
            sv = plsc.load_gather(s_vm, [sidx[sl]])
            dv = plsc.load_gather(d_vm, [didx[sl]])
            e = sv + dv
            e = jnp.where(e >= 0.0, e, e * jnp.float32(0.2))
            ex = jnp.exp(e)
            ex_vm[sl] = ex
            plsc.addupdate_scatter(denom_vm, [didx[sl]], ex)
            return carry2

        lax.fori_loop(0, CH // L, _v16, 0)
        gather.wait()

        def _scale(i, carry2):
            exb = plsc.load_gather(ex_vm, [jnp.full((L,), i, jnp.int32)])
            for j in range(D // L):
                sl = pl.ds(j * L, L)
                rows_vm[i, sl] = rows_vm[i, sl] * exb
            return carry2

        lax.fori_loop(0, CH, _scale, 0)
        pltpu.sync_copy(rows_vm, acc_sh.at[didx], add=True)
        return carry

    lax.fori_loop(0, EPT_P // CH, _chunk, 0)

    pltpu.sync_copy(denom_vm.at[pl.ds(0, N)], denom_out.at[wid])
    plsc.subcore_barrier()

    @pl.when(s == 0)
    def _():
        pltpu.sync_copy(acc_sh.at[pl.ds(0, N)], w_out.at[c])


_gat_sc = pl.kernel(
    _gat_sc_body,
    out_type=[
        jax.ShapeDtypeStruct((NC, N, D), jnp.float32),
        jax.ShapeDtypeStruct((NW, N), jnp.float32),
    ],
    mesh=_MESH,
    scratch_types=[
        pltpu.VMEM((NACC,), jnp.float32),
        pltpu.VMEM((NACC,), jnp.float32),
        pltpu.VMEM((NACC,), jnp.float32),
        pltpu.VMEM((CH,), jnp.int32),
        pltpu.VMEM((CH,), jnp.int32),
        pltpu.VMEM((CH,), jnp.float32),
        pltpu.VMEM((CH, D), jnp.float32),
        pltpu.VMEM_SHARED((NACC, D), jnp.float32),
        pltpu.SemaphoreType.DMA,
    ],
    compiler_params=_SC_PARAMS,
)


def _mpnn_sc_body(src_hbm, dst_hbm, hw_hbm, ec_hbm, zeros_hbm,
                  agg_out,
                  sidx, didx, rows_vm, ec_vm, acc_sh, sem, sem2):
    c = lax.axis_index("c")
    s = lax.axis_index("s")
    wid = c * NS + s

    @pl.when(s == 0)
    def _():
        pltpu.sync_copy(zeros_hbm, acc_sh.at[pl.ds(0, N)])

    plsc.subcore_barrier()

    base_e = wid * EPT_L

    def _chunk(g, carry):
        eb = pl.multiple_of(base_e + g * CH, CH)
        pltpu.sync_copy(src_hbm.at[pl.ds(eb, CH)], sidx)
        pltpu.sync_copy(dst_hbm.at[pl.ds(eb, CH)], didx)
        gather = pltpu.async_copy(hw_hbm.at[sidx], rows_vm, sem)
        lin = pltpu.async_copy(ec_hbm.at[pl.ds(eb, CH), :], ec_vm, sem2)
        gather.wait()
        lin.wait()

        def _relu(i, carry2):
            for j in range(D // L):
                sl = pl.ds(j * L, L)
                a = rows_vm[i, sl] + ec_vm[i, sl]
                rows_vm[i, sl] = jnp.maximum(a, jnp.float32(0.0))
            return carry2

        lax.fori_loop(0, CH, _relu, 0)
        pltpu.sync_copy(rows_vm, acc_sh.at[didx], add=True)
        return carry

    lax.fori_loop(0, EPT_L // CH, _chunk, 0)
    plsc.subcore_barrier()

    @pl.when(s == 0)
    def _():
        pltpu.sync_copy(acc_sh.at[pl.ds(0, N)], agg_out.at[c])


_mpnn_sc = pl.kernel(
    _mpnn_sc_body,
    out_type=jax.ShapeDtypeStruct((NC, N, D), jnp.float32),
    mesh=_MESH,
    scratch_types=[
        pltpu.VMEM((CH,), jnp.int32),
        pltpu.VMEM((CH,), jnp.int32),
        pltpu.VMEM((CH, D), jnp.float32),
        pltpu.VMEM((CH, D), jnp.float32),
        pltpu.VMEM_SHARED((NACC, D), jnp.float32),
        pltpu.SemaphoreType.DMA,
        pltpu.SemaphoreType.DMA,
    ],
    compiler_params=_SC_PARAMS,
)


_TM = 1000
_TE = 1024


def _ppre_body(x_ref, wg_ref, a8_ref, h_ref, sd_ref):
    h = _dot(x_ref[...], wg_ref[...])
    h_ref[...] = h
    sd_ref[...] = _dot(h, a8_ref[...])


def _protein_pre(x, w_gat, a8):
    return pl.pallas_call(
        _ppre_body,
        grid=(N // _TM,),
        in_specs=[
            pl.BlockSpec((_TM, D), lambda i: (i, 0)),
            pl.BlockSpec((D, D), lambda i: (0, 0)),
            pl.BlockSpec((D, 8), lambda i: (0, 0)),
        ],
        out_specs=[
            pl.BlockSpec((_TM, D), lambda i: (i, 0)),
            pl.BlockSpec((_TM, 8), lambda i: (i, 0)),
        ],
        out_shape=[
            jax.ShapeDtypeStruct((N, D), jnp.float32),
            jax.ShapeDtypeStruct((N, 8), jnp.float32),
        ],
    )(x, w_gat, a8)


def _lpre_body(x_ref, wa_ref, ba_ref, wmt_ref, xe_ref, hw_ref):
    xe = jnp.maximum(_dot(x_ref[...], wa_ref[...]) + ba_ref[...], 0.0)
    xe_ref[...] = xe
    hw_ref[...] = _dot(xe, wmt_ref[...])


def _ligand_pre(x, w_atom, b_atom2, wm_top):
    return pl.pallas_call(
        _lpre_body,
        grid=(N // _TM,),
        in_specs=[
            pl.BlockSpec((_TM, D), lambda i: (i, 0)),
            pl.BlockSpec((D, D), lambda i: (0, 0)),
            pl.BlockSpec((1, D), lambda i: (0, 0)),
            pl.BlockSpec((D, D), lambda i: (0, 0)),
        ],
        out_specs=[
            pl.BlockSpec((_TM, D), lambda i: (i, 0)),
            pl.BlockSpec((_TM, D), lambda i: (i, 0)),
        ],
        out_shape=[
            jax.ShapeDtypeStruct((N, D), jnp.float32),
            jax.ShapeDtypeStruct((N, D), jnp.float32),
        ],
    )(x, w_atom, b_atom2, wm_top)


def _epre_body(at_ref, wb_ref, bb_ref, wmb_ref, bm_ref, ec_ref):
    t = jnp.maximum(_dot(at_ref[...], wb_ref[...]) + bb_ref[...], 0.0)
    ec_ref[...] = _dot(t, wmb_ref[...]) + bm_ref[...]


def _edge_pre(attr_pad, w_bond, b_bond2, wm_bot, b_msg2):
    return pl.pallas_call(
        _epre_body,
        grid=(E_L_PAD // _TE,),
        in_specs=[
            pl.BlockSpec((_TE, D_EDGE), lambda i: (i, 0)),
            pl.BlockSpec((D_EDGE, D), lambda i: (0, 0)),
            pl.BlockSpec((1, D), lambda i: (0, 0)),
            pl.BlockSpec((D, D), lambda i: (0, 0)),
            pl.BlockSpec((1, D), lambda i: (0, 0)),
        ],
        out_specs=pl.BlockSpec((_TE, D), lambda i: (i, 0)),
        out_shape=jax.ShapeDtypeStruct((E_L_PAD, D), jnp.float32),
    )(attr_pad, w_bond, b_bond2, wm_bot, b_msg2)


def _upd_body(h_ref, a0_ref, a1_ref, wut_ref, wub_ref, b_ref, wmt_ref,
              h2_ref, hw2_ref):
    agg = a0_ref[...] + a1_ref[...]
    t = _dot(h_ref[...], wut_ref[...]) + _dot(agg, wub_ref[...]) + b_ref[...]
    h2 = jnp.maximum(t, 0.0)
    h2_ref[...] = h2
    hw2_ref[...] = _dot(h2, wmt_ref[...])


def _step_update(h, a0, a1, wu_top, wu_bot, b_upd2, wm_top):
    return pl.pallas_call(
        _upd_body,
        grid=(N // _TM,),
        in_specs=[
            pl.BlockSpec((_TM, D), lambda i: (i, 0)),
            pl.BlockSpec((_TM, D), lambda i: (i, 0)),
            pl.BlockSpec((_TM, D), lambda i: (i, 0)),
            pl.BlockSpec((D, D), lambda i: (0, 0)),
            pl.BlockSpec((D, D), lambda i: (0, 0)),
            pl.BlockSpec((1, D), lambda i: (0, 0)),
            pl.BlockSpec((D, D), lambda i: (0, 0)),
        ],
        out_specs=[
            pl.BlockSpec((_TM, D), lambda i: (i, 0)),
            pl.BlockSpec((_TM, D), lambda i: (i, 0)),
        ],
        out_shape=[
            jax.ShapeDtypeStruct((N, D), jnp.float32),
            jax.ShapeDtypeStruct((N, D), jnp.float32),
        ],
    )(h, a0, a1, wu_top, wu_bot, b_upd2, wm_top)


def _final_body(w0_ref, w1_ref, dp_ref, h3_ref, wi_ref, bi_ref, wk_ref,
                bk_ref, kd_ref):
    denom = jnp.sum(dp_ref[...], axis=0)[:, None]
    pn = jnp.maximum((w0_ref[...] + w1_ref[...]) / (denom + 1e-16), 0.0)
    p_repr = jnp.sum(pn, axis=0, keepdims=True) * (1.0 / N)
    l_repr = jnp.sum(h3_ref[...], axis=0, keepdims=True) * (1.0 / N)
    cat = jnp.concatenate([p_repr, l_repr], axis=1)
    inter = jnp.maximum(_dot(cat, wi_ref[...]) + bi_ref[...], 0.0)
    kd_ref[...] = _dot(inter, wk_ref[...]) + bk_ref[...]


def _final(w0, w1, denom_part, h3, w_int, b_int2, w_kd, b_kd2):
    return pl.pallas_call(
        _final_body,
        out_shape=jax.ShapeDtypeStruct((1, 1), jnp.float32),
    )(w0, w1, denom_part, h3, w_int, b_int2, w_kd, b_kd2)


def kernel(protein_x, protein_edge_index, ligand_x, ligand_edge_index,
           ligand_edge_attr, W_atom, b_atom, W_bond, b_bond, W_gat, a_src,
           a_dst, W_msg, b_msg, W_upd, b_upd, W_int, b_int, W_kd, b_kd):
    f32 = jnp.float32
    i32 = jnp.int32

    a8 = jnp.concatenate(
        [a_src[:, None], a_dst[:, None], jnp.zeros((D, 6), f32)], axis=1)
    wm_top, wm_bot = W_msg[:D], W_msg[D:]
    wu_top, wu_bot = W_upd[:D], W_upd[D:]

    src_p = jnp.concatenate(
        [protein_edge_index[0], jnp.zeros((E_P_PAD - E_P,), i32)])
    dst_p = jnp.concatenate(
        [protein_edge_index[1], jnp.full((E_P_PAD - E_P,), DUMMY, i32)])
    src_l = jnp.concatenate(
        [ligand_edge_index[0], jnp.zeros((E_L_PAD - E_L,), i32)])
    dst_l = jnp.concatenate(
        [ligand_edge_index[1], jnp.full((E_L_PAD - E_L,), DUMMY, i32)])
    attr_pad = jnp.concatenate(
        [ligand_edge_attr, jnp.zeros((E_L_PAD - E_L, D_EDGE), f32)])
    zeros_nd = jnp.zeros((N, D), f32)

    b_atom2 = b_atom[None, :]
    b_bond2 = b_bond[None, :]
    b_msg2 = b_msg[None, :]
    b_upd2 = b_upd[None, :]
    b_int2 = b_int[None, :]
    b_kd2 = b_kd[None, :]

    h_p, sd = _protein_pre(protein_x, W_gat, a8)
    s_ = jnp.asarray(sd[:, 0])
    d_ = jnp.asarray(sd[:, 1])
    w_part, denom_part = _gat_sc(src_p, dst_p, s_, d_, h_p, zeros_nd)

    x_emb, hw = _ligand_pre(ligand_x, W_atom, b_atom2, wm_top)
    ec = _edge_pre(attr_pad, W_bond, b_bond2, wm_bot, b_msg2)

    h = x_emb
    for _ in range(3):
        agg = _mpnn_sc(src_l, dst_l, hw, ec, zeros_nd)
        h, hw = _step_update(h, agg[0], agg[1], wu_top, wu_bot, b_upd2,
                             wm_top)

    kd2 = _final(w_part[0], w_part[1], denom_part, h, W_int, b_int2,
                 W_kd, b_kd2)
    return kd2[0]


# speedup vs baseline: 5.0952x; 1.0587x over previous
import functools

import jax
import jax.numpy as jnp
from jax import lax
from jax.experimental import pallas as pl
from jax.experimental.pallas import tpu as pltpu
from jax.experimental.pallas import tpu_sc as plsc

N = 10000
D = 128
D_EDGE = 16
NC, NS, L = 2, 16, 16
NW = NC * NS
CH = 128
CHL = 64
NACC = 10016
DUMMY = N

E_P = 320000
E_P_PAD = 327680
EPT_P = E_P_PAD // NW
E_L = 160000
E_L_PAD = 163840
EPT_L = E_L_PAD // NW

_HI = lax.Precision.HIGHEST


def _dot(a, b):
    return lax.dot_general(a, b, (((1,), (0,)), ((), ())), precision=_HI)


_MESH = plsc.VectorSubcoreMesh(core_axis_name="c", subcore_axis_name="s")
_SC_PARAMS = pltpu.CompilerParams(needs_layout_passes=False,
                                  use_tc_tiling_on_sc=False)


NCH_P = EPT_P // CH
NCH_L = EPT_L // CHL


def _gat_sc_body(src_hbm, dst_hbm, s_hbm, d_hbm, h_hbm, zeros_hbm,
                 w_out, denom_out,
                 s_vm, d_vm, denom_vm, sidx0, didx0, sidx1, didx1, ex_vm,
                 rows_vm, acc_sh, gsem, isem0, isem1):
    c = lax.axis_index("c")
    s = lax.axis_index("s")
    wid = c * NS + s

    @pl.when(s == 0)
    def _():
        pltpu.sync_copy(zeros_hbm, acc_sh.at[pl.ds(0, N)])

    pltpu.sync_copy(s_hbm, s_vm.at[pl.ds(0, N)])
    pltpu.sync_copy(d_hbm, d_vm.at[pl.ds(0, N)])
    s_vm[pl.ds(N, L)] = jnp.zeros((L,), jnp.float32)
    d_vm[pl.ds(N, L)] = jnp.zeros((L,), jnp.float32)

    def _zero(i, carry):
        denom_vm[pl.ds(i * L, L)] = jnp.zeros((L,), jnp.float32)
        return carry

    lax.fori_loop(0, NACC // L, _zero, 0)
    plsc.subcore_barrier()

    sidxs = (sidx0, sidx1)
    didxs = (didx0, didx1)
    isems = (isem0, isem1)
    row_base = wid * NCH_P

    def _fire_idx(b, g):
        pltpu.async_copy(src_hbm.at[row_base + g], sidxs[b], isems[b])
        pltpu.async_copy(dst_hbm.at[row_base + g], didxs[b], isems[b])

    def _drain_idx(b):
        pltpu.make_async_copy(src_hbm.at[0], sidxs[b], isems[b]).wait()
        pltpu.make_async_copy(dst_hbm.at[0], didxs[b], isems[b]).wait()

    def _compute(b, g):
        sidx, didx = sidxs[b], didxs[b]
        gather = pltpu.async_copy(h_hbm.at[sidx], rows_vm, gsem)

        def _v16(i, carry2):
            sl = pl.ds(i * L, L)
            sv = plsc.load_gather(s_vm, [sidx[sl]])
            dv = plsc.load_gather(d_vm, [didx[sl]])
            e = sv + dv
            e = jnp.where(e >= 0.0, e, e * jnp.float32(0.2))
            ex = jnp.exp(e)
            ex_vm[sl] = ex
            plsc.addupdate_scatter(denom_vm, [didx[sl]], ex)
            return carry2

        lax.fori_loop(0, CH // L, _v16, 0)
        gather.wait()

        def _scale(i, carry2):
            exb = plsc.load_gather(ex_vm, [jnp.full((L,), i, jnp.int32)])
            for j in range(D // L):
                sl = pl.ds(j * L, L)
                rows_vm[i, sl] = rows_vm[i, sl] * exb
            return carry2

        lax.fori_loop(0, CH, _scale, 0)
        pltpu.sync_copy(rows_vm, acc_sh.at[didx], add=True)

    _fire_idx(0, 0)

    def _pair(g, carry):
        c0 = 2 * g
        _fire_idx(1, c0 + 1)
        _drain_idx(0)
        _compute(0, c0)
        _fire_idx(0, jnp.minimum(c0 + 2, NCH_P - 1))
        _drain_idx(1)
        _compute(1, c0 + 1)
        return carry

    lax.fori_loop(0, NCH_P // 2, _pair, 0)
    _drain_idx(0)

    pltpu.sync_copy(denom_vm.at[pl.ds(0, N)], denom_out.at[wid])
    plsc.subcore_barrier()

    @pl.when(s == 0)
    def _():
        pltpu.sync_copy(acc_sh.at[pl.ds(0, N)], w_out.at[c])


_gat_sc = pl.kernel(
    _gat_sc_body,
    out_type=[
        jax.ShapeDtypeStruct((NC, N, D), jnp.float32),
        jax.ShapeDtypeStruct((NW, N), jnp.float32),
    ],
    mesh=_MESH,
    scratch_types=[
        pltpu.VMEM((NACC,), jnp.float32),
        pltpu.VMEM((NACC,), jnp.float32),
        pltpu.VMEM((NACC,), jnp.float32),
        pltpu.VMEM((CH,), jnp.int32),
        pltpu.VMEM((CH,), jnp.int32),
        pltpu.VMEM((CH,), jnp.int32),
        pltpu.VMEM((CH,), jnp.int32),
        pltpu.VMEM((CH,), jnp.float32),
        pltpu.VMEM((CH, D), jnp.float32),
        pltpu.VMEM_SHARED((NACC, D), jnp.float32),
        pltpu.SemaphoreType.DMA,
        pltpu.SemaphoreType.DMA,
        pltpu.SemaphoreType.DMA,
    ],
    compiler_params=_SC_PARAMS,
)


def _mpnn_sc_body(src_hbm, dst_hbm, hw_hbm, ec_hbm, zeros_hbm,
                  agg_out,
                  src_all, dst_all, rows0, rows1, ec0, ec1, acc_sh,
                  gsem0, gsem1, esem0, esem1):
    c = lax.axis_index("c")
    s = lax.axis_index("s")
    wid = c * NS + s

    @pl.when(s == 0)
    def _():
        pltpu.sync_copy(zeros_hbm, acc_sh.at[pl.ds(0, N)])

    row_base = wid * NCH_L
    pltpu.sync_copy(src_hbm.at[pl.ds(row_base, NCH_L)],
                    src_all.at[pl.ds(0, NCH_L)])
    pltpu.sync_copy(dst_hbm.at[pl.ds(row_base, NCH_L)],
                    dst_all.at[pl.ds(0, NCH_L)])
    for j in range(CHL // L):
        src_all[NCH_L, pl.ds(j * L, L)] = jnp.zeros((L,), jnp.int32)
    plsc.subcore_barrier()

    rows = (rows0, rows1)
    ecs = (ec0, ec1)
    gsems = (gsem0, gsem1)
    esems = (esem0, esem1)

    def _fire(b, g):
        pltpu.async_copy(hw_hbm.at[src_all.at[g]], rows[b], gsems[b])
        eb = pl.multiple_of(
            jnp.minimum(row_base + g, E_L_PAD // CHL - 1) * CHL, CHL)
        pltpu.async_copy(ec_hbm.at[pl.ds(eb, CHL), :], ecs[b], esems[b])

    def _drain(b):
        pltpu.make_async_copy(hw_hbm.at[src_all.at[0]], rows[b],
                              gsems[b]).wait()
        pltpu.make_async_copy(ec_hbm.at[pl.ds(0, CHL), :], ecs[b],
                              esems[b]).wait()

    def _compute(b, g):
        rv = rows[b]
        ev = ecs[b]
        _drain(b)

        def _relu(i, carry2):
            for j in range(D // L):
                sl = pl.ds(j * L, L)
                a = rv[i, sl] + ev[i, sl]
                rv[i, sl] = jnp.maximum(a, jnp.float32(0.0))
            return carry2

        lax.fori_loop(0, CHL, _relu, 0)
        pltpu.sync_copy(rv, acc_sh.at[dst_all.at[g]], add=True)

    _fire(0, 0)

    def _pair(g, carry):
        c0 = 2 * g
        _fire(1, c0 + 1)
        _compute(0, c0)
        _fire(0, c0 + 2)
        _compute(1, c0 + 1)
        return carry

    lax.fori_loop(0, NCH_L // 2, _pair, 0)
    _drain(0)
    plsc.subcore_barrier()

    @pl.when(s == 0)
    def _():
        pltpu.sync_copy(acc_sh.at[pl.ds(0, N)], agg_out.at[c])


_mpnn_sc = pl.kernel(
    _mpnn_sc_body,
    out_type=jax.ShapeDtypeStruct((NC, N, D), jnp.float32),
    mesh=_MESH,
    scratch_types=[
        pltpu.VMEM((NCH_L + 1, CHL), jnp.int32),
        pltpu.VMEM((NCH_L + 1, CHL), jnp.int32),
        pltpu.VMEM((CHL, D), jnp.float32),
        pltpu.VMEM((CHL, D), jnp.float32),
        pltpu.VMEM((CHL, D), jnp.float32),
        pltpu.VMEM((CHL, D), jnp.float32),
        pltpu.VMEM_SHARED((NACC, D), jnp.float32),
        pltpu.SemaphoreType.DMA,
        pltpu.SemaphoreType.DMA,
        pltpu.SemaphoreType.DMA,
        pltpu.SemaphoreType.DMA,
    ],
    compiler_params=_SC_PARAMS,
)


_TM = 1000
_TE = 1024


def _ppre_body(x_ref, wg_ref, a8_ref, h_ref, sd_ref):
    h = _dot(x_ref[...], wg_ref[...])
    h_ref[...] = h
    sd_ref[...] = _dot(h, a8_ref[...])


def _protein_pre(x, w_gat, a8):
    return pl.pallas_call(
        _ppre_body,
        grid=(N // _TM,),
        in_specs=[
            pl.BlockSpec((_TM, D), lambda i: (i, 0)),
            pl.BlockSpec((D, D), lambda i: (0, 0)),
            pl.BlockSpec((D, 8), lambda i: (0, 0)),
        ],
        out_specs=[
            pl.BlockSpec((_TM, D), lambda i: (i, 0)),
            pl.BlockSpec((_TM, 8), lambda i: (i, 0)),
        ],
        out_shape=[
            jax.ShapeDtypeStruct((N, D), jnp.float32),
            jax.ShapeDtypeStruct((N, 8), jnp.float32),
        ],
    )(x, w_gat, a8)


def _lpre_body(x_ref, wa_ref, ba_ref, wmt_ref, xe_ref, hw_ref):
    xe = jnp.maximum(_dot(x_ref[...], wa_ref[...]) + ba_ref[...], 0.0)
    xe_ref[...] = xe
    hw_ref[...] = _dot(xe, wmt_ref[...])


def _ligand_pre(x, w_atom, b_atom2, wm_top):
    return pl.pallas_call(
        _lpre_body,
        grid=(N // _TM,),
        in_specs=[
            pl.BlockSpec((_TM, D), lambda i: (i, 0)),
            pl.BlockSpec((D, D), lambda i: (0, 0)),
            pl.BlockSpec((1, D), lambda i: (0, 0)),
            pl.BlockSpec((D, D), lambda i: (0, 0)),
        ],
        out_specs=[
            pl.BlockSpec((_TM, D), lambda i: (i, 0)),
            pl.BlockSpec((_TM, D), lambda i: (i, 0)),
        ],
        out_shape=[
            jax.ShapeDtypeStruct((N, D), jnp.float32),
            jax.ShapeDtypeStruct((N, D), jnp.float32),
        ],
    )(x, w_atom, b_atom2, wm_top)


def _epre_body(at_ref, wb_ref, bb_ref, wmb_ref, bm_ref, ec_ref):
    t = jnp.maximum(_dot(at_ref[...], wb_ref[...]) + bb_ref[...], 0.0)
    ec_ref[...] = _dot(t, wmb_ref[...]) + bm_ref[...]


def _edge_pre(attr_pad, w_bond, b_bond2, wm_bot, b_msg2):
    return pl.pallas_call(
        _epre_body,
        grid=(E_L_PAD // _TE,),
        in_specs=[
            pl.BlockSpec((_TE, D_EDGE), lambda i: (i, 0)),
            pl.BlockSpec((D_EDGE, D), lambda i: (0, 0)),
            pl.BlockSpec((1, D), lambda i: (0, 0)),
            pl.BlockSpec((D, D), lambda i: (0, 0)),
            pl.BlockSpec((1, D), lambda i: (0, 0)),
        ],
        out_specs=pl.BlockSpec((_TE, D), lambda i: (i, 0)),
        out_shape=jax.ShapeDtypeStruct((E_L_PAD, D), jnp.float32),
    )(attr_pad, w_bond, b_bond2, wm_bot, b_msg2)


def _upd_body(h_ref, a0_ref, a1_ref, wut_ref, wub_ref, b_ref, wmt_ref,
              h2_ref, hw2_ref):
    agg = a0_ref[...] + a1_ref[...]
    t = _dot(h_ref[...], wut_ref[...]) + _dot(agg, wub_ref[...]) + b_ref[...]
    h2 = jnp.maximum(t, 0.0)
    h2_ref[...] = h2
    hw2_ref[...] = _dot(h2, wmt_ref[...])


def _step_update(h, a0, a1, wu_top, wu_bot, b_upd2, wm_top):
    return pl.pallas_call(
        _upd_body,
        grid=(N // _TM,),
        in_specs=[
            pl.BlockSpec((_TM, D), lambda i: (i, 0)),
            pl.BlockSpec((_TM, D), lambda i: (i, 0)),
            pl.BlockSpec((_TM, D), lambda i: (i, 0)),
            pl.BlockSpec((D, D), lambda i: (0, 0)),
            pl.BlockSpec((D, D), lambda i: (0, 0)),
            pl.BlockSpec((1, D), lambda i: (0, 0)),
            pl.BlockSpec((D, D), lambda i: (0, 0)),
        ],
        out_specs=[
            pl.BlockSpec((_TM, D), lambda i: (i, 0)),
            pl.BlockSpec((_TM, D), lambda i: (i, 0)),
        ],
        out_shape=[
            jax.ShapeDtypeStruct((N, D), jnp.float32),
            jax.ShapeDtypeStruct((N, D), jnp.float32),
        ],
    )(h, a0, a1, wu_top, wu_bot, b_upd2, wm_top)


def _final_body(w0_ref, w1_ref, dp_ref, h3_ref, wi_ref, bi_ref, wk_ref,
                bk_ref, kd_ref):
    denom = jnp.sum(dp_ref[...], axis=0)[:, None]
    pn = jnp.maximum((w0_ref[...] + w1_ref[...]) / (denom + 1e-16), 0.0)
    p_repr = jnp.sum(pn, axis=0, keepdims=True) * (1.0 / N)
    l_repr = jnp.sum(h3_ref[...], axis=0, keepdims=True) * (1.0 / N)
    cat = jnp.concatenate([p_repr, l_repr], axis=1)
    inter = jnp.maximum(_dot(cat, wi_ref[...]) + bi_ref[...], 0.0)
    kd_ref[...] = _dot(inter, wk_ref[...]) + bk_ref[...]


def _final(w0, w1, denom_part, h3, w_int, b_int2, w_kd, b_kd2):
    return pl.pallas_call(
        _final_body,
        out_shape=jax.ShapeDtypeStruct((1, 1), jnp.float32),
    )(w0, w1, denom_part, h3, w_int, b_int2, w_kd, b_kd2)


def kernel(protein_x, protein_edge_index, ligand_x, ligand_edge_index,
           ligand_edge_attr, W_atom, b_atom, W_bond, b_bond, W_gat, a_src,
           a_dst, W_msg, b_msg, W_upd, b_upd, W_int, b_int, W_kd, b_kd):
    f32 = jnp.float32
    i32 = jnp.int32

    a8 = jnp.concatenate(
        [a_src[:, None], a_dst[:, None], jnp.zeros((D, 6), f32)], axis=1)
    wm_top, wm_bot = W_msg[:D], W_msg[D:]
    wu_top, wu_bot = W_upd[:D], W_upd[D:]

    src_p = jnp.concatenate(
        [protein_edge_index[0], jnp.zeros((E_P_PAD - E_P,), i32)]
    ).reshape(E_P_PAD // CH, CH)
    dst_p = jnp.concatenate(
        [protein_edge_index[1], jnp.full((E_P_PAD - E_P,), DUMMY, i32)]
    ).reshape(E_P_PAD // CH, CH)
    src_l = jnp.concatenate(
        [ligand_edge_index[0], jnp.zeros((E_L_PAD - E_L,), i32)]
    ).reshape(E_L_PAD // CHL, CHL)
    dst_l = jnp.concatenate(
        [ligand_edge_index[1], jnp.full((E_L_PAD - E_L,), DUMMY, i32)]
    ).reshape(E_L_PAD // CHL, CHL)
    attr_pad = jnp.concatenate(
        [ligand_edge_attr, jnp.zeros((E_L_PAD - E_L, D_EDGE), f32)])
    zeros_nd = jnp.zeros((N, D), f32)

    b_atom2 = b_atom[None, :]
    b_bond2 = b_bond[None, :]
    b_msg2 = b_msg[None, :]
    b_upd2 = b_upd[None, :]
    b_int2 = b_int[None, :]
    b_kd2 = b_kd[None, :]

    h_p, sd = _protein_pre(protein_x, W_gat, a8)
    s_ = jnp.asarray(sd[:, 0])
    d_ = jnp.asarray(sd[:, 1])
    w_part, denom_part = _gat_sc(src_p, dst_p, s_, d_, h_p, zeros_nd)

    x_emb, hw = _ligand_pre(ligand_x, W_atom, b_atom2, wm_top)
    ec = _edge_pre(attr_pad, W_bond, b_bond2, wm_bot, b_msg2)

    h = x_emb
    for _ in range(3):
        agg = _mpnn_sc(src_l, dst_l, hw, ec, zeros_nd)
        h, hw = _step_update(h, agg[0], agg[1], wu_top, wu_bot, b_upd2,
                             wm_top)

    kd2 = _final(w_part[0], w_part[1], denom_part, h, W_int, b_int2,
                 W_kd, b_kd2)
    return kd2[0]


# speedup vs baseline: 7.0772x; 1.3890x over previous
import functools

import jax
import jax.numpy as jnp
from jax import lax
from jax.experimental import pallas as pl
from jax.experimental.pallas import tpu as pltpu
from jax.experimental.pallas import tpu_sc as plsc

N = 10000
D = 128
D_EDGE = 16
NC, NS, L = 2, 16, 16
NW = NC * NS
CH = 128
CHL = 128
NACC = 10016
DUMMY = N

E_P = 320000
E_P_PAD = 327680
EPT_P = E_P_PAD // NW
E_L = 160000
E_L_PAD = 163840
EPT_L = E_L_PAD // NW

_HI = lax.Precision.HIGHEST


def _dot(a, b):
    return lax.dot_general(a, b, (((1,), (0,)), ((), ())), precision=_HI)


_MESH = plsc.VectorSubcoreMesh(core_axis_name="c", subcore_axis_name="s")
_SC_PARAMS = pltpu.CompilerParams(needs_layout_passes=False,
                                  use_tc_tiling_on_sc=False)


NCH_P = EPT_P // CH
NCH_L = EPT_L // CHL


def _gat_sc_body(src_hbm, dst_hbm, s_hbm, d_hbm, h_hbm, zeros_hbm,
                 w_out, denom_out,
                 s_vm, d_vm, denom_vm, sidx0, didx0, sidx1, didx1, ex_vm,
                 rows_vm, acc_sh, gsem, isem0, isem1):
    c = lax.axis_index("c")
    s = lax.axis_index("s")
    wid = c * NS + s

    @pl.when(s == 0)
    def _():
        pltpu.sync_copy(zeros_hbm, acc_sh.at[pl.ds(0, N)])

    pltpu.sync_copy(s_hbm, s_vm.at[pl.ds(0, N)])
    pltpu.sync_copy(d_hbm, d_vm.at[pl.ds(0, N)])
    s_vm[pl.ds(N, L)] = jnp.zeros((L,), jnp.float32)
    d_vm[pl.ds(N, L)] = jnp.zeros((L,), jnp.float32)

    def _zero(i, carry):
        denom_vm[pl.ds(i * L, L)] = jnp.zeros((L,), jnp.float32)
        return carry

    lax.fori_loop(0, NACC // L, _zero, 0)
    plsc.subcore_barrier()

    sidxs = (sidx0, sidx1)
    didxs = (didx0, didx1)
    isems = (isem0, isem1)
    row_base = wid * NCH_P

    def _fire_idx(b, g):
        pltpu.async_copy(src_hbm.at[row_base + g], sidxs[b], isems[b])
        pltpu.async_copy(dst_hbm.at[row_base + g], didxs[b], isems[b])

    def _drain_idx(b):
        pltpu.make_async_copy(src_hbm.at[0], sidxs[b], isems[b]).wait()
        pltpu.make_async_copy(dst_hbm.at[0], didxs[b], isems[b]).wait()

    def _compute(b, g):
        sidx, didx = sidxs[b], didxs[b]
        gather = pltpu.async_copy(h_hbm.at[sidx], rows_vm, gsem)

        def _v16(i, carry2):
            sl = pl.ds(i * L, L)
            sv = plsc.load_gather(s_vm, [sidx[sl]])
            dv = plsc.load_gather(d_vm, [didx[sl]])
            e = sv + dv
            e = jnp.where(e >= 0.0, e, e * jnp.float32(0.2))
            ex = jnp.exp(e)
            ex_vm[sl] = ex
            plsc.addupdate_scatter(denom_vm, [didx[sl]], ex)
            return carry2

        lax.fori_loop(0, CH // L, _v16, 0)
        gather.wait()

        def _scale(i, carry2):
            exb = plsc.load_gather(ex_vm, [jnp.full((L,), i, jnp.int32)])
            exb2 = plsc.pack(exb, exb,
                             format=plsc.PackFormat.INTERLEAVED)
            for j in range(D // (2 * L)):
                sl = pl.ds(j * 2 * L, 2 * L)
                rows_vm[i, sl] = rows_vm[i, sl] * exb2
            return carry2

        lax.fori_loop(0, CH, _scale, 0)
        pltpu.sync_copy(rows_vm, acc_sh.at[didx], add=True)

    _fire_idx(0, 0)

    def _pair(g, carry):
        c0 = 2 * g
        _fire_idx(1, c0 + 1)
        _drain_idx(0)
        _compute(0, c0)
        _fire_idx(0, jnp.minimum(c0 + 2, NCH_P - 1))
        _drain_idx(1)
        _compute(1, c0 + 1)
        return carry

    lax.fori_loop(0, NCH_P // 2, _pair, 0)
    _drain_idx(0)

    pltpu.sync_copy(denom_vm.at[pl.ds(0, N)], denom_out.at[wid])
    plsc.subcore_barrier()

    @pl.when(s == 0)
    def _():
        pltpu.sync_copy(acc_sh.at[pl.ds(0, N)], w_out.at[c])


_gat_sc = pl.kernel(
    _gat_sc_body,
    out_type=[
        jax.ShapeDtypeStruct((NC, N, D), jnp.bfloat16),
        jax.ShapeDtypeStruct((NW, N), jnp.float32),
    ],
    mesh=_MESH,
    scratch_types=[
        pltpu.VMEM((NACC,), jnp.float32),
        pltpu.VMEM((NACC,), jnp.float32),
        pltpu.VMEM((NACC,), jnp.float32),
        pltpu.VMEM((CH,), jnp.int32),
        pltpu.VMEM((CH,), jnp.int32),
        pltpu.VMEM((CH,), jnp.int32),
        pltpu.VMEM((CH,), jnp.int32),
        pltpu.VMEM((CH,), jnp.float32),
        pltpu.VMEM((CH, D), jnp.bfloat16),
        pltpu.VMEM_SHARED((NACC, D), jnp.bfloat16),
        pltpu.SemaphoreType.DMA,
        pltpu.SemaphoreType.DMA,
        pltpu.SemaphoreType.DMA,
    ],
    compiler_params=_SC_PARAMS,
)


def _mpnn_sc_body(src_hbm, dst_hbm, hw_hbm, ec_hbm, zeros_hbm,
                  agg_out,
                  src_all, dst_all, rows0, rows1, ec0, ec1, acc_sh,
                  gsem0, gsem1, esem0, esem1):
    c = lax.axis_index("c")
    s = lax.axis_index("s")
    wid = c * NS + s

    @pl.when(s == 0)
    def _():
        pltpu.sync_copy(zeros_hbm, acc_sh.at[pl.ds(0, N)])

    row_base = wid * NCH_L
    pltpu.sync_copy(src_hbm.at[pl.ds(row_base, NCH_L)],
                    src_all.at[pl.ds(0, NCH_L)])
    pltpu.sync_copy(dst_hbm.at[pl.ds(row_base, NCH_L)],
                    dst_all.at[pl.ds(0, NCH_L)])
    for j in range(CHL // L):
        src_all[NCH_L, pl.ds(j * L, L)] = jnp.zeros((L,), jnp.int32)
    plsc.subcore_barrier()

    rows = (rows0, rows1)
    ecs = (ec0, ec1)
    gsems = (gsem0, gsem1)
    esems = (esem0, esem1)

    def _fire(b, g):
        pltpu.async_copy(hw_hbm.at[src_all.at[g]], rows[b], gsems[b])
        eb = pl.multiple_of(
            jnp.minimum(row_base + g, E_L_PAD // CHL - 1) * CHL, CHL)
        pltpu.async_copy(ec_hbm.at[pl.ds(eb, CHL), :], ecs[b], esems[b])

    def _drain(b):
        pltpu.make_async_copy(hw_hbm.at[src_all.at[0]], rows[b],
                              gsems[b]).wait()
        pltpu.make_async_copy(ec_hbm.at[pl.ds(0, CHL), :], ecs[b],
                              esems[b]).wait()

    def _compute(b, g):
        rv = rows[b]
        ev = ecs[b]
        _drain(b)

        zero32 = jnp.zeros((2 * L,), jnp.bfloat16)

        def _relu(i, carry2):
            for j in range(D // (2 * L)):
                sl = pl.ds(j * 2 * L, 2 * L)
                a = rv[i, sl] + ev[i, sl]
                rv[i, sl] = jnp.maximum(a, zero32)
            return carry2

        lax.fori_loop(0, CHL, _relu, 0)
        pltpu.sync_copy(rv, acc_sh.at[dst_all.at[g]], add=True)

    _fire(0, 0)

    def _pair(g, carry):
        c0 = 2 * g
        _fire(1, c0 + 1)
        _compute(0, c0)
        _fire(0, c0 + 2)
        _compute(1, c0 + 1)
        return carry

    lax.fori_loop(0, NCH_L // 2, _pair, 0)
    _drain(0)
    plsc.subcore_barrier()

    @pl.when(s == 0)
    def _():
        pltpu.sync_copy(acc_sh.at[pl.ds(0, N)], agg_out.at[c])


_mpnn_sc = pl.kernel(
    _mpnn_sc_body,
    out_type=jax.ShapeDtypeStruct((NC, N, D), jnp.bfloat16),
    mesh=_MESH,
    scratch_types=[
        pltpu.VMEM((NCH_L + 1, CHL), jnp.int32),
        pltpu.VMEM((NCH_L + 1, CHL), jnp.int32),
        pltpu.VMEM((CHL, D), jnp.bfloat16),
        pltpu.VMEM((CHL, D), jnp.bfloat16),
        pltpu.VMEM((CHL, D), jnp.bfloat16),
        pltpu.VMEM((CHL, D), jnp.bfloat16),
        pltpu.VMEM_SHARED((NACC, D), jnp.bfloat16),
        pltpu.SemaphoreType.DMA,
        pltpu.SemaphoreType.DMA,
        pltpu.SemaphoreType.DMA,
        pltpu.SemaphoreType.DMA,
    ],
    compiler_params=_SC_PARAMS,
)


_TM = 2000
_TE = 1024


def _ppre_body(x_ref, wg_ref, a8_ref, h_ref, sd_ref):
    h = _dot(x_ref[...], wg_ref[...])
    h_ref[...] = h.astype(jnp.bfloat16)
    sd_ref[...] = _dot(h, a8_ref[...])


def _protein_pre(x, w_gat, a8):
    return pl.pallas_call(
        _ppre_body,
        grid=(N // _TM,),
        in_specs=[
            pl.BlockSpec((_TM, D), lambda i: (i, 0)),
            pl.BlockSpec((D, D), lambda i: (0, 0)),
            pl.BlockSpec((D, 8), lambda i: (0, 0)),
        ],
        out_specs=[
            pl.BlockSpec((_TM, D), lambda i: (i, 0)),
            pl.BlockSpec((_TM, 8), lambda i: (i, 0)),
        ],
        out_shape=[
            jax.ShapeDtypeStruct((N, D), jnp.bfloat16),
            jax.ShapeDtypeStruct((N, 8), jnp.float32),
        ],
    )(x, w_gat, a8)


def _lpre_body(x_ref, wa_ref, ba_ref, wmt_ref, xe_ref, hw_ref):
    xe = jnp.maximum(_dot(x_ref[...], wa_ref[...]) + ba_ref[...], 0.0)
    xe_ref[...] = xe
    hw_ref[...] = _dot(xe, wmt_ref[...]).astype(jnp.bfloat16)


def _ligand_pre(x, w_atom, b_atom2, wm_top):
    return pl.pallas_call(
        _lpre_body,
        grid=(N // _TM,),
        in_specs=[
            pl.BlockSpec((_TM, D), lambda i: (i, 0)),
            pl.BlockSpec((D, D), lambda i: (0, 0)),
            pl.BlockSpec((1, D), lambda i: (0, 0)),
            pl.BlockSpec((D, D), lambda i: (0, 0)),
        ],
        out_specs=[
            pl.BlockSpec((_TM, D), lambda i: (i, 0)),
            pl.BlockSpec((_TM, D), lambda i: (i, 0)),
        ],
        out_shape=[
            jax.ShapeDtypeStruct((N, D), jnp.float32),
            jax.ShapeDtypeStruct((N, D), jnp.bfloat16),
        ],
    )(x, w_atom, b_atom2, wm_top)


def _epre_body(at_ref, wb_ref, bb_ref, wmb_ref, bm_ref, ec_ref):
    t = jnp.maximum(_dot(at_ref[...], wb_ref[...]) + bb_ref[...], 0.0)
    ec_ref[...] = (_dot(t, wmb_ref[...]) + bm_ref[...]).astype(jnp.bfloat16)


def _edge_pre(attr_pad, w_bond, b_bond2, wm_bot, b_msg2):
    return pl.pallas_call(
        _epre_body,
        grid=(E_L_PAD // _TE,),
        in_specs=[
            pl.BlockSpec((_TE, D_EDGE), lambda i: (i, 0)),
            pl.BlockSpec((D_EDGE, D), lambda i: (0, 0)),
            pl.BlockSpec((1, D), lambda i: (0, 0)),
            pl.BlockSpec((D, D), lambda i: (0, 0)),
            pl.BlockSpec((1, D), lambda i: (0, 0)),
        ],
        out_specs=pl.BlockSpec((_TE, D), lambda i: (i, 0)),
        out_shape=jax.ShapeDtypeStruct((E_L_PAD, D), jnp.bfloat16),
    )(attr_pad, w_bond, b_bond2, wm_bot, b_msg2)


def _upd_body(h_ref, a0_ref, a1_ref, wut_ref, wub_ref, b_ref, wmt_ref,
              h2_ref, hw2_ref):
    agg = (a0_ref[...].astype(jnp.float32)
           + a1_ref[...].astype(jnp.float32))
    t = _dot(h_ref[...], wut_ref[...]) + _dot(agg, wub_ref[...]) + b_ref[...]
    h2 = jnp.maximum(t, 0.0)
    h2_ref[...] = h2
    hw2_ref[...] = _dot(h2, wmt_ref[...]).astype(jnp.bfloat16)


def _step_update(h, a0, a1, wu_top, wu_bot, b_upd2, wm_top):
    return pl.pallas_call(
        _upd_body,
        grid=(N // _TM,),
        in_specs=[
            pl.BlockSpec((_TM, D), lambda i: (i, 0)),
            pl.BlockSpec((_TM, D), lambda i: (i, 0)),
            pl.BlockSpec((_TM, D), lambda i: (i, 0)),
            pl.BlockSpec((D, D), lambda i: (0, 0)),
            pl.BlockSpec((D, D), lambda i: (0, 0)),
            pl.BlockSpec((1, D), lambda i: (0, 0)),
            pl.BlockSpec((D, D), lambda i: (0, 0)),
        ],
        out_specs=[
            pl.BlockSpec((_TM, D), lambda i: (i, 0)),
            pl.BlockSpec((_TM, D), lambda i: (i, 0)),
        ],
        out_shape=[
            jax.ShapeDtypeStruct((N, D), jnp.float32),
            jax.ShapeDtypeStruct((N, D), jnp.bfloat16),
        ],
    )(h, a0, a1, wu_top, wu_bot, b_upd2, wm_top)


def _final_body(w0_ref, w1_ref, dp_ref, h3_ref, wi_ref, bi_ref, wk_ref,
                bk_ref, kd_ref):
    denom = jnp.sum(dp_ref[...], axis=0)[:, None]
    w = w0_ref[...].astype(jnp.float32) + w1_ref[...].astype(jnp.float32)
    pn = jnp.maximum(w / (denom + 1e-16), 0.0)
    p_repr = jnp.sum(pn, axis=0, keepdims=True) * (1.0 / N)
    l_repr = jnp.sum(h3_ref[...], axis=0, keepdims=True) * (1.0 / N)
    cat = jnp.concatenate([p_repr, l_repr], axis=1)
    inter = jnp.maximum(_dot(cat, wi_ref[...]) + bi_ref[...], 0.0)
    kd_ref[...] = _dot(inter, wk_ref[...]) + bk_ref[...]


def _final(w0, w1, denom_part, h3, w_int, b_int2, w_kd, b_kd2):
    return pl.pallas_call(
        _final_body,
        out_shape=jax.ShapeDtypeStruct((1, 1), jnp.float32),
    )(w0, w1, denom_part, h3, w_int, b_int2, w_kd, b_kd2)


def kernel(protein_x, protein_edge_index, ligand_x, ligand_edge_index,
           ligand_edge_attr, W_atom, b_atom, W_bond, b_bond, W_gat, a_src,
           a_dst, W_msg, b_msg, W_upd, b_upd, W_int, b_int, W_kd, b_kd):
    f32 = jnp.float32
    i32 = jnp.int32

    a8 = jnp.concatenate(
        [a_src[:, None], a_dst[:, None], jnp.zeros((D, 6), f32)], axis=1)
    wm_top, wm_bot = W_msg[:D], W_msg[D:]
    wu_top, wu_bot = W_upd[:D], W_upd[D:]

    src_p = jnp.concatenate(
        [protein_edge_index[0], jnp.zeros((E_P_PAD - E_P,), i32)]
    ).reshape(E_P_PAD // CH, CH)
    dst_p = jnp.concatenate(
        [protein_edge_index[1], jnp.full((E_P_PAD - E_P,), DUMMY, i32)]
    ).reshape(E_P_PAD // CH, CH)
    src_l = jnp.concatenate(
        [ligand_edge_index[0], jnp.zeros((E_L_PAD - E_L,), i32)]
    ).reshape(E_L_PAD // CHL, CHL)
    dst_l = jnp.concatenate(
        [ligand_edge_index[1], jnp.full((E_L_PAD - E_L,), DUMMY, i32)]
    ).reshape(E_L_PAD // CHL, CHL)
    attr_pad = jnp.concatenate(
        [ligand_edge_attr, jnp.zeros((E_L_PAD - E_L, D_EDGE), f32)])
    zeros_nd = jnp.zeros((N, D), jnp.bfloat16)

    b_atom2 = b_atom[None, :]
    b_bond2 = b_bond[None, :]
    b_msg2 = b_msg[None, :]
    b_upd2 = b_upd[None, :]
    b_int2 = b_int[None, :]
    b_kd2 = b_kd[None, :]

    h_p, sd = _protein_pre(protein_x, W_gat, a8)
    s_ = jnp.asarray(sd[:, 0])
    d_ = jnp.asarray(sd[:, 1])
    w_part, denom_part = _gat_sc(src_p, dst_p, s_, d_, h_p, zeros_nd)

    x_emb, hw = _ligand_pre(ligand_x, W_atom, b_atom2, wm_top)
    ec = _edge_pre(attr_pad, W_bond, b_bond2, wm_bot, b_msg2)

    h = x_emb
    for _ in range(3):
        agg = _mpnn_sc(src_l, dst_l, hw, ec, zeros_nd)
        h, hw = _step_update(h, agg[0], agg[1], wu_top, wu_bot, b_upd2,
                             wm_top)

    kd2 = _final(w_part[0], w_part[1], denom_part, h, W_int, b_int2,
                 W_kd, b_kd2)
    return kd2[0]


# speedup vs baseline: 7.5569x; 1.0678x over previous
import functools

import jax
import jax.numpy as jnp
from jax import lax
from jax.experimental import pallas as pl
from jax.experimental.pallas import tpu as pltpu
from jax.experimental.pallas import tpu_sc as plsc

N = 10000
D = 128
D_EDGE = 16
NC, NS, L = 2, 16, 16
NW = NC * NS
CH = 128
CHL = 128
NACC = 10016
DUMMY = N

E_P = 320000
E_P_PAD = 327680
EPT_P = E_P_PAD // NW
E_L = 160000
E_L_PAD = 163840
EPT_L = E_L_PAD // NW

_HI = lax.Precision.HIGHEST


def _dot(a, b):
    return lax.dot_general(a, b, (((1,), (0,)), ((), ())), precision=_HI)


_MESH = plsc.VectorSubcoreMesh(core_axis_name="c", subcore_axis_name="s")
_SC_PARAMS = pltpu.CompilerParams(needs_layout_passes=False,
                                  use_tc_tiling_on_sc=False)


NCH_P = EPT_P // CH
NCH_L = EPT_L // CHL


def _gat_sc_body(src_hbm, dst_hbm, s_hbm, d_hbm, h_hbm, zeros_hbm,
                 w_out, denom_out,
                 s_vm, d_vm, denom_vm, sidx0, didx0, sidx1, didx1, ex_vm,
                 rows0, rows1, acc_sh, gsem0, gsem1, isem0, isem1):
    c = lax.axis_index("c")
    s = lax.axis_index("s")
    wid = c * NS + s

    @pl.when(s == 0)
    def _():
        pltpu.sync_copy(zeros_hbm, acc_sh.at[pl.ds(0, N)])

    pltpu.sync_copy(s_hbm, s_vm.at[pl.ds(0, N)])
    pltpu.sync_copy(d_hbm, d_vm.at[pl.ds(0, N)])
    s_vm[pl.ds(N, L)] = jnp.zeros((L,), jnp.float32)
    d_vm[pl.ds(N, L)] = jnp.zeros((L,), jnp.float32)

    def _zero(i, carry):
        denom_vm[pl.ds(i * L, L)] = jnp.zeros((L,), jnp.float32)
        return carry

    lax.fori_loop(0, NACC // L, _zero, 0)
    plsc.subcore_barrier()

    sidxs = (sidx0, sidx1)
    didxs = (didx0, didx1)
    isems = (isem0, isem1)
    row_base = wid * NCH_P

    def _fire_idx(b, g):
        pltpu.async_copy(src_hbm.at[row_base + g], sidxs[b], isems[b])
        pltpu.async_copy(dst_hbm.at[row_base + g], didxs[b], isems[b])

    def _drain_idx(b):
        pltpu.make_async_copy(src_hbm.at[0], sidxs[b], isems[b]).wait()
        pltpu.make_async_copy(dst_hbm.at[0], didxs[b], isems[b]).wait()

    rowss = (rows0, rows1)
    gsems = (gsem0, gsem1)

    def _fire_rows(b):
        pltpu.async_copy(h_hbm.at[sidxs[b]], rowss[b], gsems[b])

    def _drain_rows(b):
        pltpu.make_async_copy(h_hbm.at[sidxs[b]], rowss[b],
                              gsems[b]).wait()

    def _scalar_phase(b):
        sidx, didx = sidxs[b], didxs[b]

        def _v16(i, carry2):
            sl = pl.ds(i * L, L)
            sv = plsc.load_gather(s_vm, [sidx[sl]])
            dv = plsc.load_gather(d_vm, [didx[sl]])
            e = sv + dv
            e = jnp.where(e >= 0.0, e, e * jnp.float32(0.2))
            ex = jnp.exp(e)
            ex_vm[sl] = ex
            plsc.addupdate_scatter(denom_vm, [didx[sl]], ex)
            return carry2

        lax.fori_loop(0, CH // L, _v16, 0)

    def _scale_scatter(b):
        rv = rowss[b]
        _drain_rows(b)

        def _scale(i, carry2):
            exb = plsc.load_gather(ex_vm, [jnp.full((L,), i, jnp.int32)])
            exb2 = plsc.pack(exb, exb,
                             format=plsc.PackFormat.INTERLEAVED)
            for j in range(D // (2 * L)):
                sl = pl.ds(j * 2 * L, 2 * L)
                rv[i, sl] = rv[i, sl] * exb2
            return carry2

        lax.fori_loop(0, CH, _scale, 0)
        pltpu.sync_copy(rv, acc_sh.at[didxs[b]], add=True)

    _fire_idx(0, 0)
    _drain_idx(0)
    _fire_rows(0)
    _fire_idx(1, 1)

    def _pair(g, carry):
        c0 = 2 * g
        _scalar_phase(0)
        _drain_idx(1)
        _fire_rows(1)
        _scale_scatter(0)
        _fire_idx(0, jnp.minimum(c0 + 2, NCH_P - 1))
        _scalar_phase(1)
        _drain_idx(0)
        _fire_rows(0)
        _scale_scatter(1)
        _fire_idx(1, jnp.minimum(c0 + 3, NCH_P - 1))
        return carry

    lax.fori_loop(0, NCH_P // 2, _pair, 0)
    _drain_idx(1)
    _drain_rows(0)

    pltpu.sync_copy(denom_vm.at[pl.ds(0, N)], denom_out.at[wid])
    plsc.subcore_barrier()

    @pl.when(s == 0)
    def _():
        pltpu.sync_copy(acc_sh.at[pl.ds(0, N)], w_out.at[c])


_gat_sc = pl.kernel(
    _gat_sc_body,
    out_type=[
        jax.ShapeDtypeStruct((NC, N, D), jnp.bfloat16),
        jax.ShapeDtypeStruct((NW, N), jnp.float32),
    ],
    mesh=_MESH,
    scratch_types=[
        pltpu.VMEM((NACC,), jnp.float32),
        pltpu.VMEM((NACC,), jnp.float32),
        pltpu.VMEM((NACC,), jnp.float32),
        pltpu.VMEM((CH,), jnp.int32),
        pltpu.VMEM((CH,), jnp.int32),
        pltpu.VMEM((CH,), jnp.int32),
        pltpu.VMEM((CH,), jnp.int32),
        pltpu.VMEM((CH,), jnp.float32),
        pltpu.VMEM((CH, D), jnp.bfloat16),
        pltpu.VMEM((CH, D), jnp.bfloat16),
        pltpu.VMEM_SHARED((NACC, D), jnp.bfloat16),
        pltpu.SemaphoreType.DMA,
        pltpu.SemaphoreType.DMA,
        pltpu.SemaphoreType.DMA,
        pltpu.SemaphoreType.DMA,
    ],
    compiler_params=_SC_PARAMS,
)


def _mpnn_sc_body(src_hbm, dst_hbm, hw_hbm, ec_hbm, zeros_hbm,
                  agg_out,
                  src_all, dst_all, rows0, rows1, ec0, ec1, acc_sh,
                  gsem0, gsem1, esem0, esem1):
    c = lax.axis_index("c")
    s = lax.axis_index("s")
    wid = c * NS + s

    @pl.when(s == 0)
    def _():
        pltpu.sync_copy(zeros_hbm, acc_sh.at[pl.ds(0, N)])

    row_base = wid * NCH_L
    pltpu.sync_copy(src_hbm.at[pl.ds(row_base, NCH_L)],
                    src_all.at[pl.ds(0, NCH_L)])
    pltpu.sync_copy(dst_hbm.at[pl.ds(row_base, NCH_L)],
                    dst_all.at[pl.ds(0, NCH_L)])
    for j in range(CHL // L):
        src_all[NCH_L, pl.ds(j * L, L)] = jnp.zeros((L,), jnp.int32)
    plsc.subcore_barrier()

    rows = (rows0, rows1)
    ecs = (ec0, ec1)
    gsems = (gsem0, gsem1)
    esems = (esem0, esem1)

    def _fire(b, g):
        pltpu.async_copy(hw_hbm.at[src_all.at[g]], rows[b], gsems[b])
        eb = pl.multiple_of(
            jnp.minimum(row_base + g, E_L_PAD // CHL - 1) * CHL, CHL)
        pltpu.async_copy(ec_hbm.at[pl.ds(eb, CHL), :], ecs[b], esems[b])

    def _drain(b):
        pltpu.make_async_copy(hw_hbm.at[src_all.at[0]], rows[b],
                              gsems[b]).wait()
        pltpu.make_async_copy(ec_hbm.at[pl.ds(0, CHL), :], ecs[b],
                              esems[b]).wait()

    def _compute(b, g):
        rv = rows[b]
        ev = ecs[b]
        _drain(b)

        zero32 = jnp.zeros((2 * L,), jnp.bfloat16)

        def _relu(i, carry2):
            for j in range(D // (2 * L)):
                sl = pl.ds(j * 2 * L, 2 * L)
                a = rv[i, sl] + ev[i, sl]
                rv[i, sl] = jnp.maximum(a, zero32)
            return carry2

        lax.fori_loop(0, CHL, _relu, 0)
        pltpu.sync_copy(rv, acc_sh.at[dst_all.at[g]], add=True)

    _fire(0, 0)

    def _pair(g, carry):
        c0 = 2 * g
        _fire(1, c0 + 1)
        _compute(0, c0)
        _fire(0, c0 + 2)
        _compute(1, c0 + 1)
        return carry

    lax.fori_loop(0, NCH_L // 2, _pair, 0)
    _drain(0)
    plsc.subcore_barrier()

    @pl.when(s == 0)
    def _():
        pltpu.sync_copy(acc_sh.at[pl.ds(0, N)], agg_out.at[c])


_mpnn_sc = pl.kernel(
    _mpnn_sc_body,
    out_type=jax.ShapeDtypeStruct((NC, N, D), jnp.bfloat16),
    mesh=_MESH,
    scratch_types=[
        pltpu.VMEM((NCH_L + 1, CHL), jnp.int32),
        pltpu.VMEM((NCH_L + 1, CHL), jnp.int32),
        pltpu.VMEM((CHL, D), jnp.bfloat16),
        pltpu.VMEM((CHL, D), jnp.bfloat16),
        pltpu.VMEM((CHL, D), jnp.bfloat16),
        pltpu.VMEM((CHL, D), jnp.bfloat16),
        pltpu.VMEM_SHARED((NACC, D), jnp.bfloat16),
        pltpu.SemaphoreType.DMA,
        pltpu.SemaphoreType.DMA,
        pltpu.SemaphoreType.DMA,
        pltpu.SemaphoreType.DMA,
    ],
    compiler_params=_SC_PARAMS,
)


_TM = 2000
_TE = 1024


def _ppre_body(x_ref, wg_ref, a8_ref, h_ref, sd_ref):
    h = _dot(x_ref[...], wg_ref[...])
    h_ref[...] = h.astype(jnp.bfloat16)
    sd_ref[...] = _dot(h, a8_ref[...])


def _protein_pre(x, w_gat, a8):
    return pl.pallas_call(
        _ppre_body,
        grid=(N // _TM,),
        in_specs=[
            pl.BlockSpec((_TM, D), lambda i: (i, 0)),
            pl.BlockSpec((D, D), lambda i: (0, 0)),
            pl.BlockSpec((D, 8), lambda i: (0, 0)),
        ],
        out_specs=[
            pl.BlockSpec((_TM, D), lambda i: (i, 0)),
            pl.BlockSpec((_TM, 8), lambda i: (i, 0)),
        ],
        out_shape=[
            jax.ShapeDtypeStruct((N, D), jnp.bfloat16),
            jax.ShapeDtypeStruct((N, 8), jnp.float32),
        ],
    )(x, w_gat, a8)


def _lpre_body(x_ref, wa_ref, ba_ref, wmt_ref, xe_ref, hw_ref):
    xe = jnp.maximum(_dot(x_ref[...], wa_ref[...]) + ba_ref[...], 0.0)
    xe_ref[...] = xe
    hw_ref[...] = _dot(xe, wmt_ref[...]).astype(jnp.bfloat16)


def _ligand_pre(x, w_atom, b_atom2, wm_top):
    return pl.pallas_call(
        _lpre_body,
        grid=(N // _TM,),
        in_specs=[
            pl.BlockSpec((_TM, D), lambda i: (i, 0)),
            pl.BlockSpec((D, D), lambda i: (0, 0)),
            pl.BlockSpec((1, D), lambda i: (0, 0)),
            pl.BlockSpec((D, D), lambda i: (0, 0)),
        ],
        out_specs=[
            pl.BlockSpec((_TM, D), lambda i: (i, 0)),
            pl.BlockSpec((_TM, D), lambda i: (i, 0)),
        ],
        out_shape=[
            jax.ShapeDtypeStruct((N, D), jnp.float32),
            jax.ShapeDtypeStruct((N, D), jnp.bfloat16),
        ],
    )(x, w_atom, b_atom2, wm_top)


def _epre_body(at_ref, wb_ref, bb_ref, wmb_ref, bm_ref, ec_ref):
    t = jnp.maximum(_dot(at_ref[...], wb_ref[...]) + bb_ref[...], 0.0)
    ec_ref[...] = (_dot(t, wmb_ref[...]) + bm_ref[...]).astype(jnp.bfloat16)


def _edge_pre(attr_pad, w_bond, b_bond2, wm_bot, b_msg2):
    return pl.pallas_call(
        _epre_body,
        grid=(E_L_PAD // _TE,),
        in_specs=[
            pl.BlockSpec((_TE, D_EDGE), lambda i: (i, 0)),
            pl.BlockSpec((D_EDGE, D), lambda i: (0, 0)),
            pl.BlockSpec((1, D), lambda i: (0, 0)),
            pl.BlockSpec((D, D), lambda i: (0, 0)),
            pl.BlockSpec((1, D), lambda i: (0, 0)),
        ],
        out_specs=pl.BlockSpec((_TE, D), lambda i: (i, 0)),
        out_shape=jax.ShapeDtypeStruct((E_L_PAD, D), jnp.bfloat16),
    )(attr_pad, w_bond, b_bond2, wm_bot, b_msg2)


def _upd_body(h_ref, a0_ref, a1_ref, wut_ref, wub_ref, b_ref, wmt_ref,
              h2_ref, hw2_ref):
    agg = (a0_ref[...].astype(jnp.float32)
           + a1_ref[...].astype(jnp.float32))
    t = _dot(h_ref[...], wut_ref[...]) + _dot(agg, wub_ref[...]) + b_ref[...]
    h2 = jnp.maximum(t, 0.0)
    h2_ref[...] = h2
    hw2_ref[...] = _dot(h2, wmt_ref[...]).astype(jnp.bfloat16)


def _step_update(h, a0, a1, wu_top, wu_bot, b_upd2, wm_top):
    return pl.pallas_call(
        _upd_body,
        grid=(N // _TM,),
        in_specs=[
            pl.BlockSpec((_TM, D), lambda i: (i, 0)),
            pl.BlockSpec((_TM, D), lambda i: (i, 0)),
            pl.BlockSpec((_TM, D), lambda i: (i, 0)),
            pl.BlockSpec((D, D), lambda i: (0, 0)),
            pl.BlockSpec((D, D), lambda i: (0, 0)),
            pl.BlockSpec((1, D), lambda i: (0, 0)),
            pl.BlockSpec((D, D), lambda i: (0, 0)),
        ],
        out_specs=[
            pl.BlockSpec((_TM, D), lambda i: (i, 0)),
            pl.BlockSpec((_TM, D), lambda i: (i, 0)),
        ],
        out_shape=[
            jax.ShapeDtypeStruct((N, D), jnp.float32),
            jax.ShapeDtypeStruct((N, D), jnp.bfloat16),
        ],
    )(h, a0, a1, wu_top, wu_bot, b_upd2, wm_top)


def _final_body(w0_ref, w1_ref, dp_ref, h3_ref, wi_ref, bi_ref, wk_ref,
                bk_ref, kd_ref):
    denom = jnp.sum(dp_ref[...], axis=0)[:, None]
    w = w0_ref[...].astype(jnp.float32) + w1_ref[...].astype(jnp.float32)
    pn = jnp.maximum(w / (denom + 1e-16), 0.0)
    p_repr = jnp.sum(pn, axis=0, keepdims=True) * (1.0 / N)
    l_repr = jnp.sum(h3_ref[...], axis=0, keepdims=True) * (1.0 / N)
    cat = jnp.concatenate([p_repr, l_repr], axis=1)
    inter = jnp.maximum(_dot(cat, wi_ref[...]) + bi_ref[...], 0.0)
    kd_ref[...] = _dot(inter, wk_ref[...]) + bk_ref[...]


def _final(w0, w1, denom_part, h3, w_int, b_int2, w_kd, b_kd2):
    return pl.pallas_call(
        _final_body,
        out_shape=jax.ShapeDtypeStruct((1, 1), jnp.float32),
    )(w0, w1, denom_part, h3, w_int, b_int2, w_kd, b_kd2)


def kernel(protein_x, protein_edge_index, ligand_x, ligand_edge_index,
           ligand_edge_attr, W_atom, b_atom, W_bond, b_bond, W_gat, a_src,
           a_dst, W_msg, b_msg, W_upd, b_upd, W_int, b_int, W_kd, b_kd):
    f32 = jnp.float32
    i32 = jnp.int32

    a8 = jnp.concatenate(
        [a_src[:, None], a_dst[:, None], jnp.zeros((D, 6), f32)], axis=1)
    wm_top, wm_bot = W_msg[:D], W_msg[D:]
    wu_top, wu_bot = W_upd[:D], W_upd[D:]

    src_p = jnp.concatenate(
        [protein_edge_index[0], jnp.zeros((E_P_PAD - E_P,), i32)]
    ).reshape(E_P_PAD // CH, CH)
    dst_p = jnp.concatenate(
        [protein_edge_index[1], jnp.full((E_P_PAD - E_P,), DUMMY, i32)]
    ).reshape(E_P_PAD // CH, CH)
    src_l = jnp.concatenate(
        [ligand_edge_index[0], jnp.zeros((E_L_PAD - E_L,), i32)]
    ).reshape(E_L_PAD // CHL, CHL)
    dst_l = jnp.concatenate(
        [ligand_edge_index[1], jnp.full((E_L_PAD - E_L,), DUMMY, i32)]
    ).reshape(E_L_PAD // CHL, CHL)
    attr_pad = jnp.concatenate(
        [ligand_edge_attr, jnp.zeros((E_L_PAD - E_L, D_EDGE), f32)])
    zeros_nd = jnp.zeros((N, D), jnp.bfloat16)

    b_atom2 = b_atom[None, :]
    b_bond2 = b_bond[None, :]
    b_msg2 = b_msg[None, :]
    b_upd2 = b_upd[None, :]
    b_int2 = b_int[None, :]
    b_kd2 = b_kd[None, :]

    h_p, sd = _protein_pre(protein_x, W_gat, a8)
    s_ = jnp.asarray(sd[:, 0])
    d_ = jnp.asarray(sd[:, 1])
    w_part, denom_part = _gat_sc(src_p, dst_p, s_, d_, h_p, zeros_nd)

    x_emb, hw = _ligand_pre(ligand_x, W_atom, b_atom2, wm_top)
    ec = _edge_pre(attr_pad, W_bond, b_bond2, wm_bot, b_msg2)

    h = x_emb
    for _ in range(3):
        agg = _mpnn_sc(src_l, dst_l, hw, ec, zeros_nd)
        h, hw = _step_update(h, agg[0], agg[1], wu_top, wu_bot, b_upd2,
                             wm_top)

    kd2 = _final(w_part[0], w_part[1], denom_part, h, W_int, b_int2,
                 W_kd, b_kd2)
    return kd2[0]


# speedup vs baseline: 8.5022x; 1.1251x over previous
import functools

import jax
import jax.numpy as jnp
from jax import lax
from jax.experimental import pallas as pl
from jax.experimental.pallas import tpu as pltpu
from jax.experimental.pallas import tpu_sc as plsc

N = 10000
D = 128
D_EDGE = 16
NC, NS, L = 2, 16, 16
NW = NC * NS
CH = 128
CHL = 128
NACC = 10016
DUMMY = N

E_P = 320000
E_P_PAD = 327680
EPT_P = E_P_PAD // NW
E_L = 160000
E_L_PAD = 163840
EPT_L = E_L_PAD // NW

_HI = lax.Precision.DEFAULT


def _dot(a, b):
    return lax.dot_general(a, b, (((1,), (0,)), ((), ())), precision=_HI)


_MESH = plsc.VectorSubcoreMesh(core_axis_name="c", subcore_axis_name="s")
_SC_PARAMS = pltpu.CompilerParams(needs_layout_passes=False,
                                  use_tc_tiling_on_sc=False)


NCH_P = EPT_P // CH
NCH_L = EPT_L // CHL


def _gat_sc_body(src_hbm, dst_hbm, s_hbm, d_hbm, h_hbm, zeros_hbm,
                 w_out, denom_out,
                 s_vm, d_vm, denom_vm, sidx0, didx0, sidx1, didx1, ex_vm,
                 rows0, rows1, acc_sh, gsem0, gsem1, isem0, isem1):
    c = lax.axis_index("c")
    s = lax.axis_index("s")
    wid = c * NS + s

    @pl.when(s == 0)
    def _():
        pltpu.sync_copy(zeros_hbm, acc_sh.at[pl.ds(0, N)])

    pltpu.sync_copy(s_hbm, s_vm.at[pl.ds(0, N)])
    pltpu.sync_copy(d_hbm, d_vm.at[pl.ds(0, N)])
    s_vm[pl.ds(N, L)] = jnp.zeros((L,), jnp.float32)
    d_vm[pl.ds(N, L)] = jnp.zeros((L,), jnp.float32)

    def _zero(i, carry):
        denom_vm[pl.ds(i * L, L)] = jnp.zeros((L,), jnp.float32)
        return carry

    lax.fori_loop(0, NACC // L, _zero, 0)
    plsc.subcore_barrier()

    sidxs = (sidx0, sidx1)
    didxs = (didx0, didx1)
    isems = (isem0, isem1)
    row_base = wid * NCH_P

    def _fire_idx(b, g):
        pltpu.async_copy(src_hbm.at[row_base + g], sidxs[b], isems[b])
        pltpu.async_copy(dst_hbm.at[row_base + g], didxs[b], isems[b])

    def _drain_idx(b):
        pltpu.make_async_copy(src_hbm.at[0], sidxs[b], isems[b]).wait()
        pltpu.make_async_copy(dst_hbm.at[0], didxs[b], isems[b]).wait()

    rowss = (rows0, rows1)
    gsems = (gsem0, gsem1)

    def _fire_rows(b):
        pltpu.async_copy(h_hbm.at[sidxs[b]], rowss[b], gsems[b])

    def _drain_rows(b):
        pltpu.make_async_copy(h_hbm.at[sidxs[b]], rowss[b],
                              gsems[b]).wait()

    def _scalar_phase(b):
        sidx, didx = sidxs[b], didxs[b]

        def _v16(i, carry2):
            sl = pl.ds(i * L, L)
            sv = plsc.load_gather(s_vm, [sidx[sl]])
            dv = plsc.load_gather(d_vm, [didx[sl]])
            e = sv + dv
            e = jnp.where(e >= 0.0, e, e * jnp.float32(0.2))
            ex = jnp.exp(e)
            ex_vm[sl] = ex
            plsc.addupdate_scatter(denom_vm, [didx[sl]], ex)
            return carry2

        lax.fori_loop(0, CH // L, _v16, 0)

    def _scale_scatter(b):
        rv = rowss[b]
        _drain_rows(b)

        def _scale(i, carry2):
            exb = plsc.load_gather(ex_vm, [jnp.full((L,), i, jnp.int32)])
            exb2 = plsc.pack(exb, exb,
                             format=plsc.PackFormat.INTERLEAVED)
            for j in range(D // (2 * L)):
                sl = pl.ds(j * 2 * L, 2 * L)
                rv[i, sl] = rv[i, sl] * exb2
            return carry2

        lax.fori_loop(0, CH, _scale, 0)
        pltpu.sync_copy(rv, acc_sh.at[didxs[b]], add=True)

    _fire_idx(0, 0)
    _drain_idx(0)
    _fire_rows(0)
    _fire_idx(1, 1)

    def _pair(g, carry):
        c0 = 2 * g
        _scalar_phase(0)
        _drain_idx(1)
        _fire_rows(1)
        _scale_scatter(0)
        _fire_idx(0, jnp.minimum(c0 + 2, NCH_P - 1))
        _scalar_phase(1)
        _drain_idx(0)
        _fire_rows(0)
        _scale_scatter(1)
        _fire_idx(1, jnp.minimum(c0 + 3, NCH_P - 1))
        return carry

    lax.fori_loop(0, NCH_P // 2, _pair, 0)
    _drain_idx(1)
    _drain_rows(0)

    pltpu.sync_copy(denom_vm.at[pl.ds(0, N)], denom_out.at[wid])
    plsc.subcore_barrier()

    @pl.when(s == 0)
    def _():
        pltpu.sync_copy(acc_sh.at[pl.ds(0, N)], w_out.at[c])


_gat_sc = pl.kernel(
    _gat_sc_body,
    out_type=[
        jax.ShapeDtypeStruct((NC, N, D), jnp.bfloat16),
        jax.ShapeDtypeStruct((NW, N), jnp.float32),
    ],
    mesh=_MESH,
    scratch_types=[
        pltpu.VMEM((NACC,), jnp.float32),
        pltpu.VMEM((NACC,), jnp.float32),
        pltpu.VMEM((NACC,), jnp.float32),
        pltpu.VMEM((CH,), jnp.int32),
        pltpu.VMEM((CH,), jnp.int32),
        pltpu.VMEM((CH,), jnp.int32),
        pltpu.VMEM((CH,), jnp.int32),
        pltpu.VMEM((CH,), jnp.float32),
        pltpu.VMEM((CH, D), jnp.bfloat16),
        pltpu.VMEM((CH, D), jnp.bfloat16),
        pltpu.VMEM_SHARED((NACC, D), jnp.bfloat16),
        pltpu.SemaphoreType.DMA,
        pltpu.SemaphoreType.DMA,
        pltpu.SemaphoreType.DMA,
        pltpu.SemaphoreType.DMA,
    ],
    compiler_params=_SC_PARAMS,
)


def _mpnn_sc_body(src_hbm, dst_hbm, hw_hbm, ec_hbm, zeros_hbm,
                  agg_out,
                  src_all, dst_all, rows0, rows1, ec0, ec1, acc_sh,
                  gsem0, gsem1, esem0, esem1):
    c = lax.axis_index("c")
    s = lax.axis_index("s")
    wid = c * NS + s

    @pl.when(s == 0)
    def _():
        pltpu.sync_copy(zeros_hbm, acc_sh.at[pl.ds(0, N)])

    row_base = wid * NCH_L
    pltpu.sync_copy(src_hbm.at[pl.ds(row_base, NCH_L)],
                    src_all.at[pl.ds(0, NCH_L)])
    pltpu.sync_copy(dst_hbm.at[pl.ds(row_base, NCH_L)],
                    dst_all.at[pl.ds(0, NCH_L)])
    for j in range(CHL // L):
        src_all[NCH_L, pl.ds(j * L, L)] = jnp.zeros((L,), jnp.int32)
    plsc.subcore_barrier()

    rows = (rows0, rows1)
    ecs = (ec0, ec1)
    gsems = (gsem0, gsem1)
    esems = (esem0, esem1)

    def _fire(b, g):
        pltpu.async_copy(hw_hbm.at[src_all.at[g]], rows[b], gsems[b])
        eb = pl.multiple_of(
            jnp.minimum(row_base + g, E_L_PAD // CHL - 1) * CHL, CHL)
        pltpu.async_copy(ec_hbm.at[pl.ds(eb, CHL), :], ecs[b], esems[b])

    def _drain(b):
        pltpu.make_async_copy(hw_hbm.at[src_all.at[0]], rows[b],
                              gsems[b]).wait()
        pltpu.make_async_copy(ec_hbm.at[pl.ds(0, CHL), :], ecs[b],
                              esems[b]).wait()

    def _compute(b, g):
        rv = rows[b]
        ev = ecs[b]
        _drain(b)

        zero32 = jnp.zeros((2 * L,), jnp.bfloat16)

        def _relu(i, carry2):
            for j in range(D // (2 * L)):
                sl = pl.ds(j * 2 * L, 2 * L)
                a = rv[i, sl] + ev[i, sl]
                rv[i, sl] = jnp.maximum(a, zero32)
            return carry2

        lax.fori_loop(0, CHL, _relu, 0)
        pltpu.sync_copy(rv, acc_sh.at[dst_all.at[g]], add=True)

    _fire(0, 0)

    def _pair(g, carry):
        c0 = 2 * g
        _fire(1, c0 + 1)
        _compute(0, c0)
        _fire(0, c0 + 2)
        _compute(1, c0 + 1)
        return carry

    lax.fori_loop(0, NCH_L // 2, _pair, 0)
    _drain(0)
    plsc.subcore_barrier()

    @pl.when(s == 0)
    def _():
        pltpu.sync_copy(acc_sh.at[pl.ds(0, N)], agg_out.at[c])


_mpnn_sc = pl.kernel(
    _mpnn_sc_body,
    out_type=jax.ShapeDtypeStruct((NC, N, D), jnp.bfloat16),
    mesh=_MESH,
    scratch_types=[
        pltpu.VMEM((NCH_L + 1, CHL), jnp.int32),
        pltpu.VMEM((NCH_L + 1, CHL), jnp.int32),
        pltpu.VMEM((CHL, D), jnp.bfloat16),
        pltpu.VMEM((CHL, D), jnp.bfloat16),
        pltpu.VMEM((CHL, D), jnp.bfloat16),
        pltpu.VMEM((CHL, D), jnp.bfloat16),
        pltpu.VMEM_SHARED((NACC, D), jnp.bfloat16),
        pltpu.SemaphoreType.DMA,
        pltpu.SemaphoreType.DMA,
        pltpu.SemaphoreType.DMA,
        pltpu.SemaphoreType.DMA,
    ],
    compiler_params=_SC_PARAMS,
)


_TM = 2000
_TE = 1024


def _ppre_body(x_ref, wg_ref, a8_ref, h_ref, sd_ref):
    h = _dot(x_ref[...], wg_ref[...])
    h_ref[...] = h.astype(jnp.bfloat16)
    sd_ref[...] = _dot(h, a8_ref[...])


def _protein_pre(x, w_gat, a8):
    return pl.pallas_call(
        _ppre_body,
        grid=(N // _TM,),
        in_specs=[
            pl.BlockSpec((_TM, D), lambda i: (i, 0)),
            pl.BlockSpec((D, D), lambda i: (0, 0)),
            pl.BlockSpec((D, 8), lambda i: (0, 0)),
        ],
        out_specs=[
            pl.BlockSpec((_TM, D), lambda i: (i, 0)),
            pl.BlockSpec((_TM, 8), lambda i: (i, 0)),
        ],
        out_shape=[
            jax.ShapeDtypeStruct((N, D), jnp.bfloat16),
            jax.ShapeDtypeStruct((N, 8), jnp.float32),
        ],
    )(x, w_gat, a8)


def _lpre_body(x_ref, wa_ref, ba_ref, wmt_ref, xe_ref, hw_ref):
    xe = jnp.maximum(_dot(x_ref[...], wa_ref[...]) + ba_ref[...], 0.0)
    xe_ref[...] = xe
    hw_ref[...] = _dot(xe, wmt_ref[...]).astype(jnp.bfloat16)


def _ligand_pre(x, w_atom, b_atom2, wm_top):
    return pl.pallas_call(
        _lpre_body,
        grid=(N // _TM,),
        in_specs=[
            pl.BlockSpec((_TM, D), lambda i: (i, 0)),
            pl.BlockSpec((D, D), lambda i: (0, 0)),
            pl.BlockSpec((1, D), lambda i: (0, 0)),
            pl.BlockSpec((D, D), lambda i: (0, 0)),
        ],
        out_specs=[
            pl.BlockSpec((_TM, D), lambda i: (i, 0)),
            pl.BlockSpec((_TM, D), lambda i: (i, 0)),
        ],
        out_shape=[
            jax.ShapeDtypeStruct((N, D), jnp.float32),
            jax.ShapeDtypeStruct((N, D), jnp.bfloat16),
        ],
    )(x, w_atom, b_atom2, wm_top)


def _epre_body(at_ref, wb_ref, bb_ref, wmb_ref, bm_ref, ec_ref):
    t = jnp.maximum(_dot(at_ref[...], wb_ref[...]) + bb_ref[...], 0.0)
    ec_ref[...] = (_dot(t, wmb_ref[...]) + bm_ref[...]).astype(jnp.bfloat16)


def _edge_pre(attr_pad, w_bond, b_bond2, wm_bot, b_msg2):
    return pl.pallas_call(
        _epre_body,
        grid=(E_L_PAD // _TE,),
        in_specs=[
            pl.BlockSpec((_TE, D_EDGE), lambda i: (i, 0)),
            pl.BlockSpec((D_EDGE, D), lambda i: (0, 0)),
            pl.BlockSpec((1, D), lambda i: (0, 0)),
            pl.BlockSpec((D, D), lambda i: (0, 0)),
            pl.BlockSpec((1, D), lambda i: (0, 0)),
        ],
        out_specs=pl.BlockSpec((_TE, D), lambda i: (i, 0)),
        out_shape=jax.ShapeDtypeStruct((E_L_PAD, D), jnp.bfloat16),
    )(attr_pad, w_bond, b_bond2, wm_bot, b_msg2)


def _upd_body(h_ref, a0_ref, a1_ref, wut_ref, wub_ref, b_ref, wmt_ref,
              h2_ref, hw2_ref):
    agg = (a0_ref[0].astype(jnp.float32)
           + a1_ref[0].astype(jnp.float32))
    t = _dot(h_ref[...], wut_ref[...]) + _dot(agg, wub_ref[...]) + b_ref[...]
    h2 = jnp.maximum(t, 0.0)
    h2_ref[...] = h2
    hw2_ref[...] = _dot(h2, wmt_ref[...]).astype(jnp.bfloat16)


def _step_update(h, agg, wu_top, wu_bot, b_upd2, wm_top):
    return pl.pallas_call(
        _upd_body,
        grid=(N // _TM,),
        in_specs=[
            pl.BlockSpec((_TM, D), lambda i: (i, 0)),
            pl.BlockSpec((1, _TM, D), lambda i: (0, i, 0)),
            pl.BlockSpec((1, _TM, D), lambda i: (1, i, 0)),
            pl.BlockSpec((D, D), lambda i: (0, 0)),
            pl.BlockSpec((D, D), lambda i: (0, 0)),
            pl.BlockSpec((1, D), lambda i: (0, 0)),
            pl.BlockSpec((D, D), lambda i: (0, 0)),
        ],
        out_specs=[
            pl.BlockSpec((_TM, D), lambda i: (i, 0)),
            pl.BlockSpec((_TM, D), lambda i: (i, 0)),
        ],
        out_shape=[
            jax.ShapeDtypeStruct((N, D), jnp.float32),
            jax.ShapeDtypeStruct((N, D), jnp.bfloat16),
        ],
    )(h, agg, agg, wu_top, wu_bot, b_upd2, wm_top)


def _final_body(w_ref, dp_ref, h3_ref, wi_ref, bi_ref, wk_ref,
                bk_ref, kd_ref):
    denom = jnp.sum(dp_ref[...], axis=0)[:, None]
    w = w_ref[0].astype(jnp.float32) + w_ref[1].astype(jnp.float32)
    pn = jnp.maximum(w / (denom + 1e-16), 0.0)
    p_repr = jnp.sum(pn, axis=0, keepdims=True) * (1.0 / N)
    l_repr = jnp.sum(h3_ref[...], axis=0, keepdims=True) * (1.0 / N)
    cat = jnp.concatenate([p_repr, l_repr], axis=1)
    inter = jnp.maximum(_dot(cat, wi_ref[...]) + bi_ref[...], 0.0)
    kd_ref[...] = _dot(inter, wk_ref[...]) + bk_ref[...]


def _final(w_part, denom_part, h3, w_int, b_int2, w_kd, b_kd2):
    return pl.pallas_call(
        _final_body,
        out_shape=jax.ShapeDtypeStruct((1, 1), jnp.float32),
    )(w_part, denom_part, h3, w_int, b_int2, w_kd, b_kd2)


def kernel(protein_x, protein_edge_index, ligand_x, ligand_edge_index,
           ligand_edge_attr, W_atom, b_atom, W_bond, b_bond, W_gat, a_src,
           a_dst, W_msg, b_msg, W_upd, b_upd, W_int, b_int, W_kd, b_kd):
    f32 = jnp.float32
    i32 = jnp.int32

    a8 = jnp.concatenate(
        [a_src[:, None], a_dst[:, None], jnp.zeros((D, 6), f32)], axis=1)
    wm_top, wm_bot = W_msg[:D], W_msg[D:]
    wu_top, wu_bot = W_upd[:D], W_upd[D:]

    src_p = jnp.concatenate(
        [protein_edge_index[0], jnp.zeros((E_P_PAD - E_P,), i32)]
    ).reshape(E_P_PAD // CH, CH)
    dst_p = jnp.concatenate(
        [protein_edge_index[1], jnp.full((E_P_PAD - E_P,), DUMMY, i32)]
    ).reshape(E_P_PAD // CH, CH)
    src_l = jnp.concatenate(
        [ligand_edge_index[0], jnp.zeros((E_L_PAD - E_L,), i32)]
    ).reshape(E_L_PAD // CHL, CHL)
    dst_l = jnp.concatenate(
        [ligand_edge_index[1], jnp.full((E_L_PAD - E_L,), DUMMY, i32)]
    ).reshape(E_L_PAD // CHL, CHL)
    attr_pad = jnp.concatenate(
        [ligand_edge_attr, jnp.zeros((E_L_PAD - E_L, D_EDGE), f32)])
    zeros_nd = jnp.zeros((N, D), jnp.bfloat16)

    b_atom2 = b_atom[None, :]
    b_bond2 = b_bond[None, :]
    b_msg2 = b_msg[None, :]
    b_upd2 = b_upd[None, :]
    b_int2 = b_int[None, :]
    b_kd2 = b_kd[None, :]

    h_p, sd = _protein_pre(protein_x, W_gat, a8)
    s_ = jnp.asarray(sd[:, 0])
    d_ = jnp.asarray(sd[:, 1])
    w_part, denom_part = _gat_sc(src_p, dst_p, s_, d_, h_p, zeros_nd)

    x_emb, hw = _ligand_pre(ligand_x, W_atom, b_atom2, wm_top)
    ec = _edge_pre(attr_pad, W_bond, b_bond2, wm_bot, b_msg2)

    h = x_emb
    for _ in range(3):
        agg = _mpnn_sc(src_l, dst_l, hw, ec, zeros_nd)
        h, hw = _step_update(h, agg, wu_top, wu_bot, b_upd2, wm_top)

    kd2 = _final(w_part, denom_part, h, W_int, b_int2, W_kd, b_kd2)
    return kd2[0]
